# baseline (speedup 1.0000x reference)
import functools

import jax
import jax.numpy as jnp
from jax import lax
from jax.experimental import pallas as pl
from jax.experimental.pallas import tpu as pltpu
from jax.experimental.pallas import tpu_sc as plsc

_BATCH = 4096
_OFF = 20
_D = 64
_HC = 32
_NW = 32
_BAGS_W = _BATCH // _NW
_CB = 32
_NCH = _BAGS_W // _CB
_RPC = _CB * _OFF
_G = 128
_NG = _RPC // _G


def _tree_sum(vals):
    while len(vals) > 1:
        nxt = [vals[i] + vals[i + 1] for i in range(0, len(vals) - 1, 2)]
        if len(vals) % 2:
            nxt.append(vals[-1])
        vals = nxt
    return vals[0]


def _make_kernel():
    mesh = plsc.VectorSubcoreMesh(core_axis_name="c", subcore_axis_name="s")

    @functools.partial(
        pl.kernel,
        mesh=mesh,
        out_type=jax.ShapeDtypeStruct((_BATCH, _D), jnp.float32),
        scratch_types=[
            pltpu.VMEM((_BAGS_W * _OFF,), jnp.int32),
            pltpu.VMEM((_RPC, _HC), jnp.float32),
            pltpu.VMEM((_RPC, _HC), jnp.float32),
            pltpu.VMEM((_RPC, _HC), jnp.float32),
            pltpu.VMEM((_RPC, _HC), jnp.float32),
            pltpu.VMEM((_CB, _D), jnp.float32),
            pltpu.VMEM((_CB, _D), jnp.float32),
            pltpu.SemaphoreType.DMA,
            pltpu.SemaphoreType.DMA,
            pltpu.SemaphoreType.DMA,
            pltpu.SemaphoreType.DMA,
        ],
        compiler_params=pltpu.CompilerParams(use_tc_tiling_on_sc=False),
    )
    def emb_bag(ta, tb, idx1d, out, idx_v, ra0, ra1, rb0, rb1, out0, out1,
                gsem0, gsem1, osem0, osem1):
        rowsa = (ra0, ra1)
        rowsb = (rb0, rb1)
        outb = (out0, out1)
        gsem = (gsem0, gsem1)
        osem = (osem0, osem1)

        w = lax.axis_index("s") * 2 + lax.axis_index("c")
        pltpu.sync_copy(idx1d.at[pl.ds(w * (_BAGS_W * _OFF), _BAGS_W * _OFF)],
                        idx_v)

        def fire(ci):
            sem = gsem[ci % 2]
            cps = []
            for tbl, bufs in ((ta, rowsa), (tb, rowsb)):
                buf = bufs[ci % 2]
                cps += [
                    pltpu.async_copy(
                        tbl.at[idx_v.at[pl.ds((ci * _NG + j) * _G, _G)]],
                        buf.at[pl.ds(j * _G, _G)],
                        sem,
                    )
                    for j in range(_NG)
                ]
            return cps

        out_cp = [None, None]
        pending = fire(0)
        for ci in range(_NCH):
            nxt = fire(ci + 1) if ci + 1 < _NCH else []
            for cp in pending:
                cp.wait()
            pending = nxt

            bufa = rowsa[ci % 2]
            bufb = rowsb[ci % 2]
            ob = outb[ci % 2]
            if out_cp[ci % 2] is not None:
                out_cp[ci % 2].wait()

            def bag_body(b, carry, bufa=bufa, bufb=bufb, ob=ob):
                r0 = b * _OFF
                for half, buf in ((0, bufa), (1, bufb)):
                    for c in range(_HC // 16):
                        vals = [
                            buf[r0 + r, pl.ds(c * 16, 16)]
                            for r in range(_OFF)
                        ]
                        ob[b, pl.ds(half * _HC + c * 16, 16)] = (
                            _tree_sum(vals)
                        )
                return carry

            lax.fori_loop(0, _CB, bag_body, 0, unroll=2)

            bag0 = w * _BAGS_W + ci * _CB
            out_cp[ci % 2] = pltpu.async_copy(
                ob, out.at[pl.ds(bag0, _CB)], osem[ci % 2]
            )
        for cp in out_cp:
            if cp is not None:
                cp.wait()

    return emb_bag


_EMB_BAG = _make_kernel()


@jax.jit
def kernel(sparse_index_group_batch, sparse_offset_group_batch, weight):
    del sparse_offset_group_batch
    idx1d = sparse_index_group_batch.astype(jnp.int32)
    return _EMB_BAG(weight[:, :_HC], weight[:, _HC:], idx1d)


# speedup vs baseline: 1.7934x; 1.7934x over previous
import functools

import jax
import jax.numpy as jnp
from jax import lax
from jax.experimental import pallas as pl
from jax.experimental.pallas import tpu as pltpu
from jax.experimental.pallas import tpu_sc as plsc

_BATCH = 4096
_OFF = 20
_D = 64
_NW = 32
_BAGS_W = _BATCH // _NW
_CB = 32
_NCH = _BAGS_W // _CB
_RPC = _CB * _OFF
_G = 128
_NG = _RPC // _G


def _tree_sum(vals):
    while len(vals) > 1:
        nxt = [vals[i] + vals[i + 1] for i in range(0, len(vals) - 1, 2)]
        if len(vals) % 2:
            nxt.append(vals[-1])
        vals = nxt
    return vals[0]


def _make_kernel():
    mesh = plsc.VectorSubcoreMesh(core_axis_name="c", subcore_axis_name="s")

    @functools.partial(
        pl.kernel,
        mesh=mesh,
        out_type=jax.ShapeDtypeStruct((_BATCH, _D), jnp.float32),
        scratch_types=[
            pltpu.VMEM((_BAGS_W * _OFF,), jnp.int32),
            pltpu.VMEM((_RPC, _D), jnp.float32),
            pltpu.VMEM((_RPC, _D), jnp.float32),
            pltpu.VMEM((_CB, _D), jnp.float32),
            pltpu.VMEM((_CB, _D), jnp.float32),
            pltpu.SemaphoreType.DMA,
            pltpu.SemaphoreType.DMA,
            pltpu.SemaphoreType.DMA,
            pltpu.SemaphoreType.DMA,
        ],
        compiler_params=pltpu.CompilerParams(use_tc_tiling_on_sc=False),
    )
    def emb_bag(table, idx1d, out, idx_v, rows0, rows1, out0, out1,
                gsem0, gsem1, osem0, osem1):
        rows = (rows0, rows1)
        outb = (out0, out1)
        gsem = (gsem0, gsem1)
        osem = (osem0, osem1)

        w = lax.axis_index("s") * 2 + lax.axis_index("c")
        pltpu.sync_copy(idx1d.at[pl.ds(w * (_BAGS_W * _OFF), _BAGS_W * _OFF)],
                        idx_v)

        def fire(ci):
            buf, sem = rows[ci % 2], gsem[ci % 2]
            return [
                pltpu.async_copy(
                    table.at[idx_v.at[pl.ds((ci * _NG + j) * _G, _G)]],
                    buf.at[pl.ds(j * _G, _G)],
                    sem,
                )
                for j in range(_NG)
            ]

        out_cp = [None, None]
        pending = fire(0)
        for ci in range(_NCH):
            nxt = fire(ci + 1) if ci + 1 < _NCH else []
            for cp in pending:
                cp.wait()
            pending = nxt

            buf = rows[ci % 2]
            ob = outb[ci % 2]
            if out_cp[ci % 2] is not None:
                out_cp[ci % 2].wait()

            def bag_body(b, carry, buf=buf, ob=ob):
                r0 = b * _OFF
                for c in range(_D // 16):
                    vals = [
                        buf[r0 + r, pl.ds(c * 16, 16)] for r in range(_OFF)
                    ]
                    ob[b, pl.ds(c * 16, 16)] = _tree_sum(vals)
                return carry

            lax.fori_loop(0, _CB, bag_body, 0, unroll=4)

            bag0 = w * _BAGS_W + ci * _CB
            out_cp[ci % 2] = pltpu.async_copy(
                ob, out.at[pl.ds(bag0, _CB)], osem[ci % 2]
            )
        for cp in out_cp:
            if cp is not None:
                cp.wait()

    return emb_bag


_EMB_BAG = _make_kernel()


@jax.jit
def kernel(sparse_index_group_batch, sparse_offset_group_batch, weight):
    del sparse_offset_group_batch
    idx1d = sparse_index_group_batch.astype(jnp.int32)
    return _EMB_BAG(weight, idx1d)


# speedup vs baseline: 1.8238x; 1.0170x over previous
import functools

import jax
import jax.numpy as jnp
from jax import lax
from jax.experimental import pallas as pl
from jax.experimental.pallas import tpu as pltpu
from jax.experimental.pallas import tpu_sc as plsc

_BATCH = 4096
_OFF = 20
_D = 64
_NW = 32
_BAGS_W = _BATCH // _NW
_CB = 32
_NCH = _BAGS_W // _CB
_RPC = _CB * _OFF
_G = 128
_NG = _RPC // _G


def _tree_sum(vals):
    while len(vals) > 1:
        nxt = [vals[i] + vals[i + 1] for i in range(0, len(vals) - 1, 2)]
        if len(vals) % 2:
            nxt.append(vals[-1])
        vals = nxt
    return vals[0]


def _make_kernel():
    mesh = plsc.VectorSubcoreMesh(core_axis_name="c", subcore_axis_name="s")

    @functools.partial(
        pl.kernel,
        mesh=mesh,
        out_type=jax.ShapeDtypeStruct((_BATCH, _D), jnp.float32),
        scratch_types=[
            pltpu.VMEM((_BAGS_W * _OFF,), jnp.int32),
            pltpu.VMEM((_RPC, _D), jnp.float32),
            pltpu.VMEM((_RPC, _D), jnp.float32),
            pltpu.VMEM((_CB, _D), jnp.float32),
            pltpu.VMEM((_CB, _D), jnp.float32),
            pltpu.SemaphoreType.DMA,
            pltpu.SemaphoreType.DMA,
            pltpu.SemaphoreType.DMA,
            pltpu.SemaphoreType.DMA,
        ],
        compiler_params=pltpu.CompilerParams(use_tc_tiling_on_sc=False),
    )
    def emb_bag(table, idx1d, out, idx_v, rows0, rows1, out0, out1,
                gsem0, gsem1, osem0, osem1):
        rows = (rows0, rows1)
        outb = (out0, out1)
        gsem = (gsem0, gsem1)
        osem = (osem0, osem1)

        w = lax.axis_index("s") * 2 + lax.axis_index("c")
        pltpu.sync_copy(idx1d.at[pl.ds(w * (_BAGS_W * _OFF), _BAGS_W * _OFF)],
                        idx_v)

        def fire(ci):
            buf, sem = rows[ci % 2], gsem[ci % 2]
            return [
                pltpu.async_copy(
                    table.at[idx_v.at[pl.ds((ci * _NG + j) * _G, _G)]],
                    buf.at[pl.ds(j * _G, _G)],
                    sem,
                )
                for j in range(_NG)
            ]

        out_cp = [None, None]
        pending = fire(0)
        for ci in range(_NCH):
            nxt = fire(ci + 1) if ci + 1 < _NCH else []
            for cp in pending:
                cp.wait()
            pending = nxt

            buf = rows[ci % 2]
            ob = outb[ci % 2]
            if out_cp[ci % 2] is not None:
                out_cp[ci % 2].wait()

            def bag_body(b, carry, buf=buf, ob=ob):
                r0 = b * _OFF
                for c in range(_D // 16):
                    vals = [
                        buf[r0 + r, pl.ds(c * 16, 16)] for r in range(_OFF)
                    ]
                    ob[b, pl.ds(c * 16, 16)] = _tree_sum(vals)
                return carry

            lax.fori_loop(0, _CB, bag_body, 0, unroll=2)

            bag0 = w * _BAGS_W + ci * _CB
            out_cp[ci % 2] = pltpu.async_copy(
                ob, out.at[pl.ds(bag0, _CB)], osem[ci % 2]
            )
        for cp in out_cp:
            if cp is not None:
                cp.wait()

    return emb_bag


_EMB_BAG = _make_kernel()


@jax.jit
def kernel(sparse_index_group_batch, sparse_offset_group_batch, weight):
    del sparse_offset_group_batch
    idx1d = sparse_index_group_batch.astype(jnp.int32)
    return _EMB_BAG(weight, idx1d)


# speedup vs baseline: 1.8430x; 1.0105x over previous
import functools

import jax
import jax.numpy as jnp
from jax import lax
from jax.experimental import pallas as pl
from jax.experimental.pallas import tpu as pltpu
from jax.experimental.pallas import tpu_sc as plsc

_BATCH = 4096
_OFF = 20
_D = 64
_NW = 32
_BAGS_W = _BATCH // _NW
_CB = 32
_NCH = _BAGS_W // _CB
_RPC = _CB * _OFF
_G = 128
_NG = _RPC // _G


def _tree_sum(vals):
    while len(vals) > 1:
        nxt = [vals[i] + vals[i + 1] for i in range(0, len(vals) - 1, 2)]
        if len(vals) % 2:
            nxt.append(vals[-1])
        vals = nxt
    return vals[0]


def _make_kernel():
    mesh = plsc.VectorSubcoreMesh(core_axis_name="c", subcore_axis_name="s")

    @functools.partial(
        pl.kernel,
        mesh=mesh,
        out_type=jax.ShapeDtypeStruct((_BATCH, _D), jnp.float32),
        scratch_types=[
            pltpu.VMEM((_BAGS_W * _OFF,), jnp.int32),
            pltpu.VMEM((_RPC, _D), jnp.float32),
            pltpu.VMEM((_RPC, _D), jnp.float32),
            pltpu.VMEM((_CB, _D), jnp.float32),
            pltpu.VMEM((_CB, _D), jnp.float32),
            pltpu.SemaphoreType.DMA,
            pltpu.SemaphoreType.DMA,
            pltpu.SemaphoreType.DMA,
            pltpu.SemaphoreType.DMA,
        ],
        compiler_params=pltpu.CompilerParams(use_tc_tiling_on_sc=False),
    )
    def emb_bag(table, idx1d, out, idx_v, rows0, rows1, out0, out1,
                gsem0, gsem1, osem0, osem1):
        rows = (rows0, rows1)
        outb = (out0, out1)
        gsem = (gsem0, gsem1)
        osem = (osem0, osem1)

        w = lax.axis_index("s") * 2 + lax.axis_index("c")
        pltpu.sync_copy(idx1d.at[pl.ds(w * (_BAGS_W * _OFF), _BAGS_W * _OFF)],
                        idx_v)

        def fire(ci):
            buf, sem = rows[ci % 2], gsem[ci % 2]
            return [
                pltpu.async_copy(
                    table.at[idx_v.at[pl.ds((ci * _NG + j) * _G, _G)]],
                    buf.at[pl.ds(j * _G, _G)],
                    sem,
                )
                for j in range(_NG)
            ]

        out_cp = [None, None]
        pending = fire(0)
        for ci in range(_NCH):
            nxt = fire(ci + 1) if ci + 1 < _NCH else []
            for cp in pending:
                cp.wait()
            pending = nxt

            buf = rows[ci % 2]
            ob = outb[ci % 2]
            if out_cp[ci % 2] is not None:
                out_cp[ci % 2].wait()

            def bag_body(b, carry, buf=buf, ob=ob):
                r0 = b * _OFF
                for cp in range(_D // 32):
                    vals0, vals1 = [], []
                    for r in range(_OFF):
                        vals0.append(buf[r0 + r, pl.ds(cp * 32, 16)])
                        vals1.append(buf[r0 + r, pl.ds(cp * 32 + 16, 16)])
                    ob[b, pl.ds(cp * 32, 16)] = _tree_sum(vals0)
                    ob[b, pl.ds(cp * 32 + 16, 16)] = _tree_sum(vals1)
                return carry

            lax.fori_loop(0, _CB, bag_body, 0, unroll=2)

            bag0 = w * _BAGS_W + ci * _CB
            out_cp[ci % 2] = pltpu.async_copy(
                ob, out.at[pl.ds(bag0, _CB)], osem[ci % 2]
            )
        for cp in out_cp:
            if cp is not None:
                cp.wait()

    return emb_bag


_EMB_BAG = _make_kernel()


@jax.jit
def kernel(sparse_index_group_batch, sparse_offset_group_batch, weight):
    del sparse_offset_group_batch
    idx1d = sparse_index_group_batch.astype(jnp.int32)
    return _EMB_BAG(weight, idx1d)


# speedup vs baseline: 1.8640x; 1.0114x over previous
import functools

import jax
import jax.numpy as jnp
from jax import lax
from jax.experimental import pallas as pl
from jax.experimental.pallas import tpu as pltpu
from jax.experimental.pallas import tpu_sc as plsc

_BATCH = 4096
_OFF = 20
_D = 64
_NW = 32
_BAGS_W = _BATCH // _NW
_CB = 32
_NCH = _BAGS_W // _CB
_RPC = _CB * _OFF
_G = 128
_NG = _RPC // _G


def _tree_sum(vals):
    while len(vals) > 1:
        nxt = [vals[i] + vals[i + 1] for i in range(0, len(vals) - 1, 2)]
        if len(vals) % 2:
            nxt.append(vals[-1])
        vals = nxt
    return vals[0]


def _make_kernel():
    mesh = plsc.VectorSubcoreMesh(core_axis_name="c", subcore_axis_name="s")

    @functools.partial(
        pl.kernel,
        mesh=mesh,
        out_type=jax.ShapeDtypeStruct((_BATCH, _D), jnp.float32),
        scratch_types=[
            pltpu.VMEM((_BAGS_W * _OFF,), jnp.int32),
            pltpu.VMEM((_RPC, _D), jnp.float32),
            pltpu.VMEM((_RPC, _D), jnp.float32),
            pltpu.VMEM((_CB, _D), jnp.float32),
            pltpu.VMEM((_CB, _D), jnp.float32),
            pltpu.SemaphoreType.DMA,
            pltpu.SemaphoreType.DMA,
            pltpu.SemaphoreType.DMA,
            pltpu.SemaphoreType.DMA,
        ],
        compiler_params=pltpu.CompilerParams(use_tc_tiling_on_sc=False),
    )
    def emb_bag(table, idx1d, out, idx_v, rows0, rows1, out0, out1,
                gsem0, gsem1, osem0, osem1):
        rows = (rows0, rows1)
        outb = (out0, out1)
        gsem = (gsem0, gsem1)
        osem = (osem0, osem1)

        w = lax.axis_index("s") * 2 + lax.axis_index("c")
        pltpu.sync_copy(idx1d.at[pl.ds(w * (_BAGS_W * _OFF), _BAGS_W * _OFF)],
                        idx_v)

        def fire(ci):
            buf, sem = rows[ci % 2], gsem[ci % 2]
            return [
                pltpu.async_copy(
                    table.at[idx_v.at[pl.ds((ci * _NG + j) * _G, _G)]],
                    buf.at[pl.ds(j * _G, _G)],
                    sem,
                )
                for j in range(_NG)
            ]

        out_cp = [None, None]
        pending = fire(0)
        for ci in range(_NCH):
            nxt = fire(ci + 1) if ci + 1 < _NCH else []
            for cp in pending:
                cp.wait()
            pending = nxt

            buf = rows[ci % 2]
            ob = outb[ci % 2]
            if out_cp[ci % 2] is not None:
                out_cp[ci % 2].wait()

            def bag_body(b, carry, buf=buf, ob=ob):
                r0 = b * _OFF
                vals = [[], [], [], []]
                for r in range(_OFF):
                    for c in range(4):
                        vals[c].append(buf[r0 + r, pl.ds(c * 16, 16)])
                for c in range(4):
                    ob[b, pl.ds(c * 16, 16)] = _tree_sum(vals[c])
                return carry

            lax.fori_loop(0, _CB, bag_body, 0, unroll=2)

            bag0 = w * _BAGS_W + ci * _CB
            out_cp[ci % 2] = pltpu.async_copy(
                ob, out.at[pl.ds(bag0, _CB)], osem[ci % 2]
            )
        for cp in out_cp:
            if cp is not None:
                cp.wait()

    return emb_bag


_EMB_BAG = _make_kernel()


@jax.jit
def kernel(sparse_index_group_batch, sparse_offset_group_batch, weight):
    del sparse_offset_group_batch
    idx1d = sparse_index_group_batch.astype(jnp.int32)
    return _EMB_BAG(weight, idx1d)
